# baseline (speedup 1.0000x reference)
import functools

import jax
import jax.numpy as jnp
from jax.experimental import pallas as pl

NUM_CLASSES = 9
B_EV, S_EV = 8, 1280
N = B_EV * S_EV
IN_DIM, EMB, WIDTH, KNN, SDIM, PDIM, NCONV = 34, 128, 126, 32, 4, 32, 2

ROWS_EMB = 640
ROWS_GN = 320
ROWS_HEAD = 640


def _elu(x):
    return jnp.where(x > 0, x, jnp.exp(jnp.minimum(x, 0.0)) - 1.0)


def _layernorm(x, g, b, eps=1e-5):
    m = x.mean(-1, keepdims=True)
    v = jnp.mean((x - m) ** 2, axis=-1, keepdims=True)
    return (x - m) / jnp.sqrt(v + eps) * g + b


def _mm(a, b):
    return jnp.dot(a, b, preferred_element_type=jnp.float32)


def _embed_body(x_ref, w0, b0, w1, b1, w2, b2, w3, b3, o_ref):
    e = x_ref[...]
    e = _elu(_mm(e, w0[...]) + b0[...])
    e = _elu(_mm(e, w1[...]) + b1[...])
    e = _elu(_mm(e, w2[...]) + b2[...])
    o_ref[...] = _mm(e, w3[...]) + b3[...]


def _run_embed(x, nn0):
    ws = []
    for i in range(4):
        ws.append(nn0[i]["W"])
        ws.append(nn0[i]["b"].reshape(1, -1))
    full = lambda w: pl.BlockSpec(w.shape, lambda i: (0,) * w.ndim)
    return pl.pallas_call(
        _embed_body,
        grid=(N // ROWS_EMB,),
        in_specs=[pl.BlockSpec((ROWS_EMB, IN_DIM), lambda i: (i, 0))]
        + [full(w) for w in ws],
        out_specs=pl.BlockSpec((ROWS_EMB, EMB), lambda i: (i, 0)),
        out_shape=jax.ShapeDtypeStruct((N, EMB), jnp.float32),
    )(x, *ws)


def _gravnet_body(xr_ref, xf_ref, ws, bs_row, wh, bh_row,
                  w1, w2a, w2b, b2, g_ref, be_ref, o_ref):
    xr = xr_ref[0, 0]
    xf = xf_ref[0, 0]

    s_rows = _mm(xr, ws[0]) + bs_row[0]
    s_full = _mm(xf, ws[0]) + bs_row[0]
    s_cols = s_full.T
    h = _mm(xf, wh[0]) + bh_row[0]
    ht = h.T

    d2 = jnp.zeros((ROWS_GN, S_EV), jnp.float32)
    for c in range(SDIM):
        diff = s_rows[:, c:c + 1] - s_cols[c:c + 1, :]
        d2 = d2 + diff * diff

    iota = jax.lax.broadcasted_iota(jnp.int32, (ROWS_GN, S_EV), 1)

    keys = (jax.lax.bitcast_convert_type(d2, jnp.int32) & ~0x7FF) | iota

    def _bit(i, p):
        q = p + (jnp.int32(1) << (30 - i))
        cnt = jnp.count_nonzero(keys < q, axis=1, keepdims=True)
        return jnp.where(cnt >= KNN, p, q)

    kth = jax.lax.fori_loop(0, 31, _bit, jnp.zeros((ROWS_GN, 1), jnp.int32),
                            unroll=31)
    mask = keys <= kth
    a = jnp.where(mask, jnp.exp(-10.0 * d2), 0.0)
    mean_agg = _mm(a, h) * (1.0 / KNN)

    ab = a.astype(jnp.bfloat16)
    htb = ht.astype(jnp.bfloat16)
    negb = jnp.where(mask, 0.0, -1e6).astype(jnp.bfloat16)
    cols = []
    for c in range(PDIM):
        prod = ab * htb[c:c + 1, :] + negb
        cols.append(jnp.max(prod, axis=1, keepdims=True).astype(jnp.float32))
    max_agg = jnp.concatenate(cols, axis=1)

    x_new = _mm(xr, w1[0]) + _mm(mean_agg, w2a[0]) \
        + _mm(max_agg, w2b[0]) + b2[0]
    o_ref[0, 0] = _layernorm(xr + x_new, g_ref[0], be_ref[0])


def _stack_p(p_id, p_reg, getter):
    return jnp.stack([getter(p_id), getter(p_reg)])


def _run_gravnet_pair(x4, p_id, p_reg):
    args = []
    for get in (
        lambda p: p["lin_s"]["W"],
        lambda p: p["lin_s"]["b"].reshape(1, SDIM),
        lambda p: p["lin_h"]["W"],
        lambda p: p["lin_h"]["b"].reshape(1, PDIM),
        lambda p: p["lin_out1"]["W"],
        lambda p: p["lin_out2"]["W"][:PDIM],
        lambda p: p["lin_out2"]["W"][PDIM:],
        lambda p: p["lin_out2"]["b"].reshape(1, EMB),
        lambda p: p["norm1"]["g"].reshape(1, EMB),
        lambda p: p["norm1"]["b"].reshape(1, EMB),
    ):
        args.append(_stack_p(p_id, p_reg, get))
    wspec = lambda w: pl.BlockSpec((1,) + w.shape[1:],
                                   lambda s, e, r: (s,) + (0,) * (w.ndim - 1))
    return pl.pallas_call(
        _gravnet_body,
        grid=(2, B_EV, S_EV // ROWS_GN),
        in_specs=[
            pl.BlockSpec((1, 1, ROWS_GN, EMB), lambda s, e, r: (s, e, r, 0)),
            pl.BlockSpec((1, 1, S_EV, EMB), lambda s, e, r: (s, e, 0, 0)),
        ] + [wspec(w) for w in args],
        out_specs=pl.BlockSpec((1, 1, ROWS_GN, EMB),
                               lambda s, e, r: (s, e, r, 0)),
        out_shape=jax.ShapeDtypeStruct((2, B_EV, S_EV, EMB), jnp.float32),
    )(x4, x4, *args)


def _ffn_flat(p, split_sizes):
    flat = []
    w0 = p["lin"][0]["W"]
    off = 0
    for sz in split_sizes:
        flat.append(w0[off:off + sz])
        off += sz
    flat.append(p["lin"][0]["b"].reshape(1, -1))
    flat.append(p["ln"][0]["g"].reshape(1, -1))
    flat.append(p["ln"][0]["b"].reshape(1, -1))
    for i in range(1, 4):
        flat.append(p["lin"][i]["W"])
        flat.append(p["lin"][i]["b"].reshape(1, -1))
        flat.append(p["ln"][i]["g"].reshape(1, -1))
        flat.append(p["ln"][i]["b"].reshape(1, -1))
    flat.append(p["lin"][4]["W"])
    flat.append(p["lin"][4]["b"].reshape(1, -1))
    return flat


def _apply_ffn(refs_iter, chunks, n_chunks):
    acc = None
    for _ in range(n_chunks):
        w = next(refs_iter)
        ch = chunks[_]
        part = _mm(ch, w[...])
        acc = part if acc is None else acc + part
    x = acc + next(refs_iter)[...]
    x = _elu(x)
    x = _layernorm(x, next(refs_iter)[...], next(refs_iter)[...])
    for _ in range(3):
        x = _mm(x, next(refs_iter)[...]) + next(refs_iter)[...]
        x = _elu(x)
        x = _layernorm(x, next(refs_iter)[...], next(refs_iter)[...])
    return _mm(x, next(refs_iter)[...]) + next(refs_iter)[...]


def _heads_body(n_id_refs, n_head_refs, *refs):
    (inp_ref, e_id1, e_id2, e_reg1, e_reg2) = refs[:5]
    wrefs = refs[5:-3]
    id_ref, mom_ref, chg_ref = refs[-3:]
    inp = inp_ref[...]
    it = iter(wrefs)

    id_chunks = [inp, e_id1[0], e_id2[0]]
    id_refs = [next(it) for _ in range(n_id_refs)]
    preds_id = _apply_ffn(iter(id_refs), id_chunks, 3)
    id_ref[...] = preds_id

    reg_chunks = [inp, e_reg1[0], e_reg2[0], preds_id]
    outs = []
    for _ in range(5):
        head_refs = [next(it) for _ in range(n_head_refs)]
        outs.append(_apply_ffn(iter(head_refs), reg_chunks, 4))
    pt = outs[0] + inp[:, 1:2]
    eta = outs[1] + inp[:, 2:3]
    phi = outs[2] + inp[:, 3:4]
    en = outs[3] + inp[:, 4:5]
    mom_ref[...] = jnp.concatenate([pt, eta, phi, en], axis=1)
    chg_ref[...] = outs[4]


def _run_heads(inp, pair1, pair2, params):
    id_flat = _ffn_flat(params["nn_id"], [IN_DIM, EMB, EMB])
    reg_split = [IN_DIM, EMB, EMB, NUM_CLASSES]
    reg_flats = [
        _ffn_flat(params[k], reg_split)
        for k in ("nn_pt", "nn_eta", "nn_phi", "nn_energy", "nn_charge")
    ]
    wargs = list(id_flat)
    for f in reg_flats:
        wargs.extend(f)
    full = lambda w: pl.BlockSpec(w.shape, lambda i: (0,) * w.ndim)
    row = lambda d: pl.BlockSpec((ROWS_HEAD, d), lambda i: (i, 0))
    emb_spec = lambda s: pl.BlockSpec((1, ROWS_HEAD, EMB),
                                      lambda i, _s=s: (_s, i, 0))
    body = functools.partial(_heads_body, len(id_flat), len(reg_flats[0]))
    return pl.pallas_call(
        body,
        grid=(N // ROWS_HEAD,),
        in_specs=[row(IN_DIM), emb_spec(0), emb_spec(0),
                  emb_spec(1), emb_spec(1)]
        + [full(w) for w in wargs],
        out_specs=(row(NUM_CLASSES), row(4), row(3)),
        out_shape=(
            jax.ShapeDtypeStruct((N, NUM_CLASSES), jnp.float32),
            jax.ShapeDtypeStruct((N, 4), jnp.float32),
            jax.ShapeDtypeStruct((N, 3), jnp.float32),
        ),
    )(inp, pair1, pair2, pair1, pair2, *wargs)


def kernel(x, batch_index, params):
    del batch_index
    e = _run_embed(x, params["nn0"])
    e4 = jnp.stack([e, e]).reshape(2, B_EV, S_EV, EMB)
    pair1 = _run_gravnet_pair(e4, params["conv_id"][0], params["conv_reg"][0])
    pair2 = _run_gravnet_pair(pair1, params["conv_id"][1],
                              params["conv_reg"][1])
    return _run_heads(x, pair1.reshape(2, N, EMB), pair2.reshape(2, N, EMB),
                      params)


# speedup vs baseline: 1.0846x; 1.0846x over previous
import functools

import jax
import jax.numpy as jnp
from jax.experimental import pallas as pl

NUM_CLASSES = 9
B_EV, S_EV = 8, 1280
N = B_EV * S_EV
IN_DIM, EMB, WIDTH, KNN, SDIM, PDIM, NCONV = 34, 128, 126, 32, 4, 32, 2

ROWS_EMB = 640
ROWS_GN = 320
ROWS_HEAD = 640


def _elu(x):
    return jnp.where(x > 0, x, jnp.exp(jnp.minimum(x, 0.0)) - 1.0)


def _layernorm(x, g, b, eps=1e-5):
    m = x.mean(-1, keepdims=True)
    v = jnp.mean((x - m) ** 2, axis=-1, keepdims=True)
    return (x - m) / jnp.sqrt(v + eps) * g + b


def _mm(a, b):
    return jnp.dot(a, b, preferred_element_type=jnp.float32)


def _embed_body(x_ref, w0, b0, w1, b1, w2, b2, w3, b3, o_ref):
    e = x_ref[...]
    e = _elu(_mm(e, w0[...]) + b0[...])
    e = _elu(_mm(e, w1[...]) + b1[...])
    e = _elu(_mm(e, w2[...]) + b2[...])
    o_ref[...] = _mm(e, w3[...]) + b3[...]


def _run_embed(x, nn0):
    ws = []
    for i in range(4):
        ws.append(nn0[i]["W"])
        ws.append(nn0[i]["b"].reshape(1, -1))
    full = lambda w: pl.BlockSpec(w.shape, lambda i: (0,) * w.ndim)
    return pl.pallas_call(
        _embed_body,
        grid=(N // ROWS_EMB,),
        in_specs=[pl.BlockSpec((ROWS_EMB, IN_DIM), lambda i: (i, 0))]
        + [full(w) for w in ws],
        out_specs=pl.BlockSpec((ROWS_EMB, EMB), lambda i: (i, 0)),
        out_shape=jax.ShapeDtypeStruct((N, EMB), jnp.float32),
    )(x, *ws)


def _gravnet_body(xr_ref, xf_ref, xt_ref, ws, bs_row, wst, bs_col,
                  wh, bh_row, wht, bh_col, w1, w2a, w2b, b2, g_ref, be_ref,
                  o_ref):
    xr = xr_ref[0]
    xf = xf_ref[0]
    xt = xt_ref[0]

    s_rows = _mm(xr, ws[...]) + bs_row[...]
    s_cols = _mm(wst[...], xt) + bs_col[...]
    h = _mm(xf, wh[...]) + bh_row[...]
    ht = _mm(wht[...], xt) + bh_col[...]

    d2 = jnp.zeros((ROWS_GN, S_EV), jnp.float32)
    for c in range(SDIM):
        diff = s_rows[:, c:c + 1] - s_cols[c:c + 1, :]
        d2 = d2 + diff * diff

    iota = jax.lax.broadcasted_iota(jnp.int32, (ROWS_GN, S_EV), 1)

    bmin = jnp.min(d2.reshape(ROWS_GN, S_EV // 128, 128), axis=1)
    t_ub = jnp.max(bmin, axis=1, keepdims=True)
    scale = 2048.0 / (t_ub + 1e-30)
    qi = jnp.minimum(d2 * scale, 2304.0).astype(jnp.int32)
    keys = (qi << 11) | iota

    def _bit(i, p):
        q = p + (jnp.int32(1) << (22 - i))
        cnt = jnp.count_nonzero(keys < q, axis=1, keepdims=True)
        return jnp.where(cnt >= KNN, p, q)

    kth = jax.lax.fori_loop(0, 23, _bit, jnp.zeros((ROWS_GN, 1), jnp.int32),
                            unroll=23)
    mask = keys <= kth
    a = jnp.where(mask, jnp.exp(-10.0 * d2), 0.0)
    mean_agg = _mm(a, h) * (1.0 / KNN)

    ab = a.astype(jnp.bfloat16)
    htb = ht.astype(jnp.bfloat16)
    negb = jnp.where(mask, 0.0, -1e6).astype(jnp.bfloat16)
    cols = []
    for c in range(PDIM):
        prod = ab * htb[c:c + 1, :] + negb
        cols.append(jnp.max(prod, axis=1, keepdims=True).astype(jnp.float32))
    max_agg = jnp.concatenate(cols, axis=1)

    x_new = _mm(xr, w1[...]) + _mm(mean_agg, w2a[...]) \
        + _mm(max_agg, w2b[...]) + b2[...]
    o_ref[0] = _layernorm(xr + x_new, g_ref[...], be_ref[...])


def _run_gravnet(x3, xt3, p):
    ws = p["lin_s"]["W"]
    bs = p["lin_s"]["b"]
    wh = p["lin_h"]["W"]
    bh = p["lin_h"]["b"]
    w2 = p["lin_out2"]["W"]
    args = [
        ws, bs.reshape(1, SDIM), ws.T, bs.reshape(SDIM, 1),
        wh, bh.reshape(1, PDIM), wh.T, bh.reshape(PDIM, 1),
        p["lin_out1"]["W"], w2[:PDIM], w2[PDIM:],
        p["lin_out2"]["b"].reshape(1, EMB),
        p["norm1"]["g"].reshape(1, EMB), p["norm1"]["b"].reshape(1, EMB),
    ]
    full = lambda w: pl.BlockSpec(w.shape, lambda e, r: (0,) * w.ndim)
    return pl.pallas_call(
        _gravnet_body,
        grid=(B_EV, S_EV // ROWS_GN),
        in_specs=[
            pl.BlockSpec((1, ROWS_GN, EMB), lambda e, r: (e, r, 0)),
            pl.BlockSpec((1, S_EV, EMB), lambda e, r: (e, 0, 0)),
            pl.BlockSpec((1, EMB, S_EV), lambda e, r: (e, 0, 0)),
        ] + [full(w) for w in args],
        out_specs=pl.BlockSpec((1, ROWS_GN, EMB), lambda e, r: (e, r, 0)),
        out_shape=jax.ShapeDtypeStruct((B_EV, S_EV, EMB), jnp.float32),
    )(x3, x3, xt3, *args)


def _ffn_flat(p, split_sizes):
    flat = []
    w0 = p["lin"][0]["W"]
    off = 0
    for sz in split_sizes:
        flat.append(w0[off:off + sz])
        off += sz
    flat.append(p["lin"][0]["b"].reshape(1, -1))
    flat.append(p["ln"][0]["g"].reshape(1, -1))
    flat.append(p["ln"][0]["b"].reshape(1, -1))
    for i in range(1, 4):
        flat.append(p["lin"][i]["W"])
        flat.append(p["lin"][i]["b"].reshape(1, -1))
        flat.append(p["ln"][i]["g"].reshape(1, -1))
        flat.append(p["ln"][i]["b"].reshape(1, -1))
    flat.append(p["lin"][4]["W"])
    flat.append(p["lin"][4]["b"].reshape(1, -1))
    return flat


def _apply_ffn(refs_iter, chunks, n_chunks):
    acc = None
    for _ in range(n_chunks):
        w = next(refs_iter)
        ch = chunks[_]
        part = _mm(ch, w[...])
        acc = part if acc is None else acc + part
    x = acc + next(refs_iter)[...]
    x = _elu(x)
    x = _layernorm(x, next(refs_iter)[...], next(refs_iter)[...])
    for _ in range(3):
        x = _mm(x, next(refs_iter)[...]) + next(refs_iter)[...]
        x = _elu(x)
        x = _layernorm(x, next(refs_iter)[...], next(refs_iter)[...])
    return _mm(x, next(refs_iter)[...]) + next(refs_iter)[...]


def _heads_body(n_id_refs, n_head_refs, *refs):
    (inp_ref, e_id1, e_id2, e_reg1, e_reg2) = refs[:5]
    wrefs = refs[5:-3]
    id_ref, mom_ref, chg_ref = refs[-3:]
    inp = inp_ref[...]
    it = iter(wrefs)

    id_chunks = [inp, e_id1[...], e_id2[...]]
    id_refs = [next(it) for _ in range(n_id_refs)]
    preds_id = _apply_ffn(iter(id_refs), id_chunks, 3)
    id_ref[...] = preds_id

    reg_chunks = [inp, e_reg1[...], e_reg2[...], preds_id]
    outs = []
    for _ in range(5):
        head_refs = [next(it) for _ in range(n_head_refs)]
        outs.append(_apply_ffn(iter(head_refs), reg_chunks, 4))
    pt = outs[0] + inp[:, 1:2]
    eta = outs[1] + inp[:, 2:3]
    phi = outs[2] + inp[:, 3:4]
    en = outs[3] + inp[:, 4:5]
    mom_ref[...] = jnp.concatenate([pt, eta, phi, en], axis=1)
    chg_ref[...] = outs[4]


def _run_heads(inp, e_id1, e_id2, e_reg1, e_reg2, params):
    id_flat = _ffn_flat(params["nn_id"], [IN_DIM, EMB, EMB])
    reg_split = [IN_DIM, EMB, EMB, NUM_CLASSES]
    reg_flats = [
        _ffn_flat(params[k], reg_split)
        for k in ("nn_pt", "nn_eta", "nn_phi", "nn_energy", "nn_charge")
    ]
    wargs = list(id_flat)
    for f in reg_flats:
        wargs.extend(f)
    full = lambda w: pl.BlockSpec(w.shape, lambda i: (0,) * w.ndim)
    row = lambda d: pl.BlockSpec((ROWS_HEAD, d), lambda i: (i, 0))
    body = functools.partial(_heads_body, len(id_flat), len(reg_flats[0]))
    return pl.pallas_call(
        body,
        grid=(N // ROWS_HEAD,),
        in_specs=[row(IN_DIM), row(EMB), row(EMB), row(EMB), row(EMB)]
        + [full(w) for w in wargs],
        out_specs=(row(NUM_CLASSES), row(4), row(3)),
        out_shape=(
            jax.ShapeDtypeStruct((N, NUM_CLASSES), jnp.float32),
            jax.ShapeDtypeStruct((N, 4), jnp.float32),
            jax.ShapeDtypeStruct((N, 3), jnp.float32),
        ),
    )(inp, e_id1, e_id2, e_reg1, e_reg2, *wargs)


def kernel(x, batch_index, params):
    del batch_index
    e = _run_embed(x, params["nn0"])

    def run_stack(convs):
        cur = e
        embs = []
        for p in convs:
            x3 = cur.reshape(B_EV, S_EV, EMB)
            xt3 = jnp.transpose(x3, (0, 2, 1))
            cur = _run_gravnet(x3, xt3, p).reshape(N, EMB)
            embs.append(cur)
        return embs

    embs_id = run_stack(params["conv_id"])
    embs_reg = run_stack(params["conv_reg"])

    return _run_heads(x, embs_id[0], embs_id[1], embs_reg[0], embs_reg[1],
                      params)
